# baseline (speedup 1.0000x reference)
import functools

import jax
import jax.numpy as jnp
from jax import lax
from jax.experimental import pallas as pl
from jax.experimental.pallas import tpu as pltpu
from jax.experimental.pallas import tpu_sc as plsc

N = 10000
E = 320000
D = 128
DO = 128

NC = 2
NS = 16
NW = NC * NS
L = 16

N_PAD = 10240
NSL = N_PAD // NS
HSL = NSL // NC

EP1 = E // NS
EC1 = 4000
NCH1 = EP1 // EC1

EP2 = E // NW
C2 = 80
NB = 4
NSB = 5
NCH_SB = EP2 // (NSB * C2)
RING = (NCH_SB // NB) * NB


def _rsqrt_nr(x):
    i = plsc.bitcast(x, jnp.int32)
    i = jnp.int32(0x5F3759DF) - (i >> 1)
    y = plsc.bitcast(i, jnp.float32)
    for _ in range(3):
        y = y * (1.5 - 0.5 * x * y * y)
    return y


def _scale_body(feat_hbm, src_hbm, dst_hbm, scaled_hbm,
                src_v, dst_v, hist_s, hist_d, shared, tmp_v, norm_v,
                acc_s, acc_d, rows_v):
    c = lax.axis_index("c")
    s = lax.axis_index("s")
    zero = jnp.zeros((L,), jnp.float32)
    ones = jnp.ones((L,), jnp.float32)

    @pl.loop(0, N_PAD // L)
    def _(i):
        hist_s[pl.ds(i * L, L)] = zero
        hist_d[pl.ds(i * L, L)] = zero

    base = s * EP1
    for k in range(NCH1):
        pltpu.sync_copy(src_hbm.at[pl.ds(base + k * EC1, EC1)], src_v)
        pltpu.sync_copy(dst_hbm.at[pl.ds(base + k * EC1, EC1)], dst_v)

        @pl.loop(0, EC1 // L)
        def _(i):
            si = src_v[pl.ds(i * L, L)]
            di = dst_v[pl.ds(i * L, L)]
            plsc.addupdate_scatter(hist_s, [si], ones)
            plsc.addupdate_scatter(hist_d, [di], ones)

    pltpu.sync_copy(hist_s, shared.at[s, 0])
    pltpu.sync_copy(hist_d, shared.at[s, 1])
    plsc.subcore_barrier()

    nbase = s * NSL

    @pl.loop(0, NSL // L)
    def _(j):
        acc_s[pl.ds(j * L, L)] = zero
        acc_d[pl.ds(j * L, L)] = zero

    @pl.loop(0, NS)
    def _(t):
        pltpu.sync_copy(shared.at[t, 0, pl.ds(nbase, NSL)], tmp_v)

        @pl.loop(0, NSL // L)
        def _(j):
            acc_s[pl.ds(j * L, L)] = acc_s[pl.ds(j * L, L)] + tmp_v[pl.ds(j * L, L)]

        pltpu.sync_copy(shared.at[t, 1, pl.ds(nbase, NSL)], tmp_v)

        @pl.loop(0, NSL // L)
        def _(j):
            acc_d[pl.ds(j * L, L)] = acc_d[pl.ds(j * L, L)] + tmp_v[pl.ds(j * L, L)]

    @pl.loop(0, NSL // L)
    def _(j):
        og = jnp.maximum(acc_s[pl.ds(j * L, L)], 1.0)
        ig = jnp.maximum(acc_d[pl.ds(j * L, L)], 1.0)
        norm_v[pl.ds(j * L, L)] = _rsqrt_nr(og * ig)

    row0 = nbase + c * HSL

    def scale_and_store(nrows):
        pltpu.sync_copy(feat_hbm.at[pl.ds(row0, nrows)],
                        rows_v.at[pl.ds(0, nrows)])

        @pl.loop(0, nrows)
        def _(r):
            v = norm_v[pl.ds(c * HSL + r, L)]
            wv = jnp.full((L,), v[0], jnp.float32)
            for j in range(D // L):
                rows_v[r, pl.ds(j * L, L)] = rows_v[r, pl.ds(j * L, L)] * wv

        pltpu.sync_copy(rows_v.at[pl.ds(0, nrows)],
                        scaled_hbm.at[pl.ds(row0, nrows)])

    TAIL = N - (N_PAD - HSL)

    @pl.when(row0 + HSL <= N)
    def _():
        scale_and_store(HSL)

    @pl.when(row0 + HSL > N)
    def _():
        scale_and_store(TAIL)


def _agg_body(scaled_hbm, srcT_hbm, dstT_hbm, agg_hbm,
              sidx, didx, rows0, rows1, rows2, rows3,
              sem0, sem1, sem2, sem3, ssem0, ssem1, ssem2, ssem3, acc_sh):
    c = lax.axis_index("c")
    s = lax.axis_index("s")
    w = s * NC + c
    bufs = [(rows0, sem0, ssem0), (rows1, sem1, ssem1),
            (rows2, sem2, ssem2), (rows3, sem3, ssem3)]

    @pl.loop(0, C2)
    def _(r):
        for j in range(D // L):
            rows0[r, pl.ds(j * L, L)] = jnp.zeros((L,), jnp.float32)

    nbase = s * NSL
    for q in range(NSL // C2):
        pltpu.sync_copy(rows0, acc_sh.at[pl.ds(nbase + q * C2, C2)])
    plsc.subcore_barrier()

    for sb in range(NSB):
        pltpu.sync_copy(srcT_hbm.at[w, sb], sidx)
        pltpu.sync_copy(dstT_hbm.at[w, sb], didx)

        for b, (rb, gs, _) in enumerate(bufs):
            pltpu.async_copy(scaled_hbm.at[sidx.at[b]], rb, gs)

        @pl.loop(0, RING // NB - 1)
        def _(i):
            k0 = i * NB
            for b, (rb, gs, ss) in enumerate(bufs):
                k = k0 + b
                pltpu.make_async_copy(
                    scaled_hbm.at[sidx.at[k]], rb, gs).wait()
                pltpu.async_copy(rb, acc_sh.at[didx.at[k]], ss, add=True)
            for b, (rb, gs, ss) in enumerate(bufs):
                k = k0 + b
                pltpu.make_async_copy(rb, acc_sh.at[didx.at[k]], ss).wait()
                pltpu.async_copy(scaled_hbm.at[sidx.at[k + NB]], rb, gs)

        ke = RING - NB
        for b, (rb, gs, ss) in enumerate(bufs):
            pltpu.make_async_copy(
                scaled_hbm.at[sidx.at[ke + b]], rb, gs).wait()
            pltpu.async_copy(rb, acc_sh.at[didx.at[ke + b]], ss, add=True)
        for b, (rb, gs, ss) in enumerate(bufs):
            pltpu.make_async_copy(rb, acc_sh.at[didx.at[ke + b]], ss).wait()
        for k in range(RING, NCH_SB):
            pltpu.async_copy(scaled_hbm.at[sidx.at[k]], rows0, sem0).wait()
            pltpu.sync_copy(rows0, acc_sh.at[didx.at[k]], add=True)

    plsc.subcore_barrier()
    pltpu.sync_copy(acc_sh.at[pl.ds(nbase, NSL)], agg_hbm.at[c, pl.ds(nbase, NSL)])


def _mm_body(agg_ref, w_ref, b_ref, o_ref):
    a = agg_ref[0] + agg_ref[1]
    o_ref[...] = (jnp.dot(a, w_ref[...], preferred_element_type=jnp.float32)
                  + b_ref[0:1, :])


_MB = 1000


def kernel(feat, edge_index, weight, bias):
    assert feat.shape == (N, D) and edge_index.shape == (2, E)
    src = edge_index[0]
    dst = edge_index[1]
    srcT = src.reshape(NW, NSB, NCH_SB, C2)
    dstT = dst.reshape(NW, NSB, NCH_SB, C2)

    mesh = plsc.VectorSubcoreMesh(core_axis_name="c", subcore_axis_name="s")

    scale_k = functools.partial(
        pl.kernel,
        out_type=jax.ShapeDtypeStruct((N, D), jnp.float32),
        mesh=mesh,
        scratch_types=[
            pltpu.VMEM((EC1,), jnp.int32),
            pltpu.VMEM((EC1,), jnp.int32),
            pltpu.VMEM((N_PAD,), jnp.float32),
            pltpu.VMEM((N_PAD,), jnp.float32),
            pltpu.VMEM_SHARED((NS, 2, N_PAD), jnp.float32),
            pltpu.VMEM((NSL,), jnp.float32),
            pltpu.VMEM((NSL + L,), jnp.float32),
            pltpu.VMEM((NSL,), jnp.float32),
            pltpu.VMEM((NSL,), jnp.float32),
            pltpu.VMEM((HSL, D), jnp.float32),
        ],
        compiler_params=pltpu.CompilerParams(needs_layout_passes=False),
    )(_scale_body)
    scaled = scale_k(feat, src, dst)

    agg_k = functools.partial(
        pl.kernel,
        out_type=jax.ShapeDtypeStruct((NC, N_PAD, D), jnp.float32),
        mesh=mesh,
        scratch_types=[
            pltpu.VMEM((NCH_SB, C2), jnp.int32),
            pltpu.VMEM((NCH_SB, C2), jnp.int32),
            pltpu.VMEM((C2, D), jnp.float32),
            pltpu.VMEM((C2, D), jnp.float32),
            pltpu.VMEM((C2, D), jnp.float32),
            pltpu.VMEM((C2, D), jnp.float32),
            pltpu.SemaphoreType.DMA,
            pltpu.SemaphoreType.DMA,
            pltpu.SemaphoreType.DMA,
            pltpu.SemaphoreType.DMA,
            pltpu.SemaphoreType.DMA,
            pltpu.SemaphoreType.DMA,
            pltpu.SemaphoreType.DMA,
            pltpu.SemaphoreType.DMA,
            pltpu.VMEM_SHARED((N_PAD, D), jnp.float32),
        ],
        compiler_params=pltpu.CompilerParams(needs_layout_passes=False),
    )(_agg_body)
    agg = agg_k(scaled, srcT, dstT)

    out = pl.pallas_call(
        _mm_body,
        grid=(N // _MB,),
        in_specs=[
            pl.BlockSpec((NC, _MB, D), lambda i: (0, i, 0)),
            pl.BlockSpec((D, DO), lambda i: (0, 0)),
            pl.BlockSpec((8, DO), lambda i: (0, 0)),
        ],
        out_specs=pl.BlockSpec((_MB, DO), lambda i: (i, 0)),
        out_shape=jax.ShapeDtypeStruct((N, DO), jnp.float32),
    )(agg, weight, jnp.broadcast_to(bias, (8, DO)))
    return out


# speedup vs baseline: 1.0311x; 1.0311x over previous
import functools

import jax
import jax.numpy as jnp
from jax import lax
from jax.experimental import pallas as pl
from jax.experimental.pallas import tpu as pltpu
from jax.experimental.pallas import tpu_sc as plsc

N = 10000
E = 320000
D = 128
DO = 128

NC = 2
NS = 16
NW = NC * NS
L = 16

N_PAD = 10240
NSL = N_PAD // NS
HSL = NSL // NC

EP1 = E // NS
EC1 = 4000
NCH1 = EP1 // EC1

EP2 = E // NW
C2 = 80
NB = 4
EP2_PAD = 10240
NSB = 4
NCH_SB = EP2_PAD // (NSB * C2)
RING = NCH_SB


def _rsqrt_nr(x):
    i = plsc.bitcast(x, jnp.int32)
    i = jnp.int32(0x5F3759DF) - (i >> 1)
    y = plsc.bitcast(i, jnp.float32)
    for _ in range(3):
        y = y * (1.5 - 0.5 * x * y * y)
    return y


def _scale_body(feat_hbm, src_hbm, dst_hbm, scaled_hbm,
                src_v, dst_v, hist_s, hist_d, shared, tmp_v, norm_v,
                acc_s, acc_d, rows_v):
    c = lax.axis_index("c")
    s = lax.axis_index("s")
    zero = jnp.zeros((L,), jnp.float32)
    ones = jnp.ones((L,), jnp.float32)

    @pl.loop(0, N_PAD // L)
    def _(i):
        hist_s[pl.ds(i * L, L)] = zero
        hist_d[pl.ds(i * L, L)] = zero

    base = s * EP1
    for k in range(NCH1):
        pltpu.sync_copy(src_hbm.at[pl.ds(base + k * EC1, EC1)], src_v)
        pltpu.sync_copy(dst_hbm.at[pl.ds(base + k * EC1, EC1)], dst_v)

        @pl.loop(0, EC1 // L)
        def _(i):
            si = src_v[pl.ds(i * L, L)]
            di = dst_v[pl.ds(i * L, L)]
            plsc.addupdate_scatter(hist_s, [si], ones)
            plsc.addupdate_scatter(hist_d, [di], ones)

    pltpu.sync_copy(hist_s, shared.at[s, 0])
    pltpu.sync_copy(hist_d, shared.at[s, 1])
    plsc.subcore_barrier()

    nbase = s * NSL

    @pl.loop(0, NSL // L)
    def _(j):
        acc_s[pl.ds(j * L, L)] = zero
        acc_d[pl.ds(j * L, L)] = zero

    @pl.loop(0, NS)
    def _(t):
        pltpu.sync_copy(shared.at[t, 0, pl.ds(nbase, NSL)], tmp_v)

        @pl.loop(0, NSL // L)
        def _(j):
            acc_s[pl.ds(j * L, L)] = acc_s[pl.ds(j * L, L)] + tmp_v[pl.ds(j * L, L)]

        pltpu.sync_copy(shared.at[t, 1, pl.ds(nbase, NSL)], tmp_v)

        @pl.loop(0, NSL // L)
        def _(j):
            acc_d[pl.ds(j * L, L)] = acc_d[pl.ds(j * L, L)] + tmp_v[pl.ds(j * L, L)]

    @pl.loop(0, NSL // L)
    def _(j):
        og = jnp.maximum(acc_s[pl.ds(j * L, L)], 1.0)
        ig = jnp.maximum(acc_d[pl.ds(j * L, L)], 1.0)
        norm_v[pl.ds(j * L, L)] = _rsqrt_nr(og * ig)

    row0 = nbase + c * HSL

    def scale_and_store(nrows):
        pltpu.sync_copy(feat_hbm.at[pl.ds(row0, nrows)],
                        rows_v.at[pl.ds(0, nrows)])

        @pl.loop(0, nrows)
        def _(r):
            v = norm_v[pl.ds(c * HSL + r, L)]
            wv = jnp.full((L,), v[0], jnp.float32)
            for j in range(D // L):
                rows_v[r, pl.ds(j * L, L)] = rows_v[r, pl.ds(j * L, L)] * wv

        pltpu.sync_copy(rows_v.at[pl.ds(0, nrows)],
                        scaled_hbm.at[pl.ds(row0, nrows)])

    TAIL = N - (N_PAD - HSL)

    @pl.when(row0 + HSL <= N)
    def _():
        scale_and_store(HSL)

    @pl.when(row0 + HSL > N)
    def _():
        scale_and_store(TAIL)


def _agg_body(scaled_hbm, srcT_hbm, dstT_hbm, agg_hbm,
              sidx, didx, rows0, rows1, rows2, rows3,
              sem0, sem1, sem2, sem3, ssem0, ssem1, ssem2, ssem3, acc_sh):
    c = lax.axis_index("c")
    s = lax.axis_index("s")
    w = s * NC + c
    bufs = [(rows0, sem0, ssem0), (rows1, sem1, ssem1),
            (rows2, sem2, ssem2), (rows3, sem3, ssem3)]

    @pl.loop(0, C2)
    def _(r):
        for j in range(D // L):
            rows0[r, pl.ds(j * L, L)] = jnp.zeros((L,), jnp.float32)

    nbase = s * NSL
    for q in range(NSL // C2):
        pltpu.sync_copy(rows0, acc_sh.at[pl.ds(nbase + q * C2, C2)])
    plsc.subcore_barrier()

    for sb in range(NSB):
        pltpu.sync_copy(srcT_hbm.at[w, sb], sidx)
        pltpu.sync_copy(dstT_hbm.at[w, sb], didx)

        for b, (rb, gs, _) in enumerate(bufs):
            pltpu.async_copy(scaled_hbm.at[sidx.at[b]], rb, gs)

        @pl.loop(0, RING // NB - 1)
        def _(i):
            k0 = i * NB
            for b, (rb, gs, ss) in enumerate(bufs):
                k = k0 + b
                pltpu.make_async_copy(
                    scaled_hbm.at[sidx.at[k]], rb, gs).wait()
                pltpu.async_copy(rb, acc_sh.at[didx.at[k]], ss, add=True)
            for b, (rb, gs, ss) in enumerate(bufs):
                k = k0 + b
                pltpu.make_async_copy(rb, acc_sh.at[didx.at[k]], ss).wait()
                pltpu.async_copy(scaled_hbm.at[sidx.at[k + NB]], rb, gs)

        ke = RING - NB
        for b, (rb, gs, ss) in enumerate(bufs):
            pltpu.make_async_copy(
                scaled_hbm.at[sidx.at[ke + b]], rb, gs).wait()
            pltpu.async_copy(rb, acc_sh.at[didx.at[ke + b]], ss, add=True)
        for b, (rb, gs, ss) in enumerate(bufs):
            pltpu.make_async_copy(rb, acc_sh.at[didx.at[ke + b]], ss).wait()

    plsc.subcore_barrier()
    pltpu.sync_copy(acc_sh.at[pl.ds(nbase, NSL)], agg_hbm.at[c, pl.ds(nbase, NSL)])


def _mm_body(agg_ref, w_ref, b_ref, o_ref):
    a = agg_ref[0] + agg_ref[1]
    o_ref[...] = (jnp.dot(a, w_ref[...], preferred_element_type=jnp.float32)
                  + b_ref[0:1, :])


_MB = 1000


def kernel(feat, edge_index, weight, bias):
    assert feat.shape == (N, D) and edge_index.shape == (2, E)
    src = edge_index[0]
    dst = edge_index[1]
    npad_e = EP2_PAD - EP2
    pad_idx = (N + (jnp.arange(npad_e, dtype=jnp.int32)[None, :]
                    + 8 * jnp.arange(NW, dtype=jnp.int32)[:, None]) % (N_PAD - N))
    srcT = jnp.concatenate([src.reshape(NW, EP2), pad_idx], axis=1)
    srcT = srcT.reshape(NW, NSB, NCH_SB, C2)
    dstT = jnp.concatenate([dst.reshape(NW, EP2), pad_idx], axis=1)
    dstT = dstT.reshape(NW, NSB, NCH_SB, C2)

    mesh = plsc.VectorSubcoreMesh(core_axis_name="c", subcore_axis_name="s")

    scale_k = functools.partial(
        pl.kernel,
        out_type=jax.ShapeDtypeStruct((N_PAD, D), jnp.float32),
        mesh=mesh,
        scratch_types=[
            pltpu.VMEM((EC1,), jnp.int32),
            pltpu.VMEM((EC1,), jnp.int32),
            pltpu.VMEM((N_PAD,), jnp.float32),
            pltpu.VMEM((N_PAD,), jnp.float32),
            pltpu.VMEM_SHARED((NS, 2, N_PAD), jnp.float32),
            pltpu.VMEM((NSL,), jnp.float32),
            pltpu.VMEM((NSL + L,), jnp.float32),
            pltpu.VMEM((NSL,), jnp.float32),
            pltpu.VMEM((NSL,), jnp.float32),
            pltpu.VMEM((HSL, D), jnp.float32),
        ],
        compiler_params=pltpu.CompilerParams(needs_layout_passes=False),
    )(_scale_body)
    scaled = scale_k(feat, src, dst)

    agg_k = functools.partial(
        pl.kernel,
        out_type=jax.ShapeDtypeStruct((NC, N_PAD, D), jnp.float32),
        mesh=mesh,
        scratch_types=[
            pltpu.VMEM((NCH_SB, C2), jnp.int32),
            pltpu.VMEM((NCH_SB, C2), jnp.int32),
            pltpu.VMEM((C2, D), jnp.float32),
            pltpu.VMEM((C2, D), jnp.float32),
            pltpu.VMEM((C2, D), jnp.float32),
            pltpu.VMEM((C2, D), jnp.float32),
            pltpu.SemaphoreType.DMA,
            pltpu.SemaphoreType.DMA,
            pltpu.SemaphoreType.DMA,
            pltpu.SemaphoreType.DMA,
            pltpu.SemaphoreType.DMA,
            pltpu.SemaphoreType.DMA,
            pltpu.SemaphoreType.DMA,
            pltpu.SemaphoreType.DMA,
            pltpu.VMEM_SHARED((N_PAD, D), jnp.float32),
        ],
        compiler_params=pltpu.CompilerParams(needs_layout_passes=False),
    )(_agg_body)
    agg = agg_k(scaled, srcT, dstT)

    out = pl.pallas_call(
        _mm_body,
        grid=(N // _MB,),
        in_specs=[
            pl.BlockSpec((NC, _MB, D), lambda i: (0, i, 0)),
            pl.BlockSpec((D, DO), lambda i: (0, 0)),
            pl.BlockSpec((8, DO), lambda i: (0, 0)),
        ],
        out_specs=pl.BlockSpec((_MB, DO), lambda i: (i, 0)),
        out_shape=jax.ShapeDtypeStruct((N, DO), jnp.float32),
    )(agg, weight, jnp.broadcast_to(bias, (8, DO)))
    return out
